# baseline (speedup 1.0000x reference)
import functools

import jax
import jax.numpy as jnp
from jax import lax
from jax.experimental import pallas as pl
from jax.experimental.pallas import tpu as pltpu
from jax.experimental.pallas import tpu_sc as plsc

N = 20000
N_PRE = 12000
N_POST = 2000
TH = 0.7
B = 128
NB = 94
PAD = NB * B
NS = 32768
SR, SL = 256, 128
NW = 32
BPW = PAD // NW
F32 = jnp.float32
I32 = jnp.int32
U32 = jnp.uint32


def _sort_body(score_ref, ord_ref):
    s = score_ref[...]
    bu = lax.bitcast_convert_type(s, U32)
    li = (lax.broadcasted_iota(I32, (SR, SL), 0) * SL
          + lax.broadcasted_iota(I32, (SR, SL), 1))
    neg = lax.bitcast_convert_type(s, I32) < 0
    ik = jnp.where(neg, bu, ~(bu ^ jnp.uint32(0x80000000)))
    ik = jnp.where(li < N, ik, jnp.uint32(0xFFFFFFFF))
    v = li

    iotaR = lax.broadcasted_iota(I32, (SR, SL), 0)
    iotaL = lax.broadcasted_iota(I32, (SR, SL), 1)

    size = 2
    while size <= NS:
        j = size // 2
        while j >= 1:
            if j >= SL:
                m = j // SL
                kd = jnp.roll(ik, -m, axis=0)
                ku = jnp.roll(ik, m, axis=0)
                vd = jnp.roll(v, -m, axis=0)
                vu = jnp.roll(v, m, axis=0)
                low = (iotaR & m) == 0
            else:
                kd = jnp.roll(ik, -j, axis=1)
                ku = jnp.roll(ik, j, axis=1)
                vd = jnp.roll(v, -j, axis=1)
                vu = jnp.roll(v, j, axis=1)
                low = (iotaL & j) == 0
            kp = jnp.where(low, kd, ku)
            vp = jnp.where(low, vd, vu)
            asc = (li & size) == 0
            take_min = low == asc
            i_is_min = (ik < kp) | ((ik == kp) & (v < vp))
            keep = i_is_min == take_min
            ik = jnp.where(keep, ik, kp)
            v = jnp.where(keep, v, vp)
            j //= 2
        size *= 2
    ord_ref[...] = v


def _gather_body(data_hbm, idx_hbm, out_hbm, idx_v, rows_v, sem):
    wid = lax.axis_index("s") * 2 + lax.axis_index("c")
    base = wid * BPW
    pltpu.sync_copy(idx_hbm.at[pl.ds(base, BPW)], idx_v)
    pltpu.async_copy(data_hbm.at[idx_v], rows_v, sem).wait()
    pltpu.sync_copy(rows_v, out_hbm.at[pl.ds(base, BPW)])


def _sc_gather(data, idx):
    k = functools.partial(
        pl.kernel,
        mesh=plsc.VectorSubcoreMesh(core_axis_name="c", subcore_axis_name="s"),
        out_type=jax.ShapeDtypeStruct((PAD, 128), F32),
        scratch_types=[
            pltpu.VMEM((BPW,), I32),
            pltpu.VMEM((BPW, 128), F32),
            pltpu.SemaphoreType.DMA,
        ],
    )(_gather_body)
    return k(data, idx)


def _decode(aref, oref, bref):
    a0, a1, a2, a3 = aref[0], aref[1], aref[2], aref[3]
    cy = (a0 + a2) / 2
    cx = (a1 + a3) / 2
    h = a2 - a0
    w = a3 - a1
    dy, dx = oref[0], oref[1]
    dh = 0.5 * jnp.exp(oref[2])
    dw = 0.5 * jnp.exp(oref[3])
    y1 = jnp.clip(cy + h * (dy - dh), 0.0, 256.0)
    x1 = jnp.clip(cx + w * (dx - dw), 0.0, 256.0)
    y2 = jnp.clip(cy + h * (dy + dh), 0.0, 256.0)
    x2 = jnp.clip(cx + w * (dx + dw), 0.0, 256.0)
    bref[0] = y1
    bref[1] = x1
    bref[2] = y2
    bref[3] = x2
    bref[4] = (y2 - y1) * (x2 - x1)


def _nms_body(ancA, offA, out, bA, dA, s_scr):
    _decode(ancA, offA, bA)

    iotaL = lax.broadcasted_iota(I32, (1, B), 1)
    iotaS = lax.broadcasted_iota(I32, (B, 1), 0)
    eye = (lax.broadcasted_iota(I32, (B, B), 0)
           == lax.broadcasted_iota(I32, (B, B), 1)).astype(F32)

    def block_step(k, count):
        def fast(cnt):
            for c in range(4):
                out[c, pl.ds(k, 1), :] = jnp.zeros((1, B), F32)
            return cnt

        def slow(cnt):
            rows = [bA[c, pl.ds(k, 1), :] for c in range(5)]
            stacked = jnp.concatenate(
                [jnp.broadcast_to(r, (B, B)) for r in rows], axis=1)
            cbs = lax.dot_general(stacked, eye, (((0,), (0,)), ((), ())),
                                  precision=lax.Precision.HIGHEST,
                                  preferred_element_type=F32)
            CB = [cbs[c * B:(c + 1) * B, :] for c in range(5)]

            def pbody(j, sup):
                cj = [dA[c, pl.ds(j, 1), :] for c in range(5)]
                tly = jnp.maximum(CB[0], cj[0])
                tlx = jnp.maximum(CB[1], cj[1])
                bry = jnp.minimum(CB[2], cj[2])
                brx = jnp.minimum(CB[3], cj[3])
                cap = jnp.where((tly < bry) & (tlx < brx),
                                (bry - tly) * (brx - tlx), 0.0)
                iou = cap / ((CB[4] + cj[4]) - cap)
                hit = jnp.max((iou >= TH).astype(F32), axis=1, keepdims=True)
                return jnp.maximum(sup, hit)

            sup_col = lax.fori_loop(0, k, pbody, jnp.zeros((B, 1), F32))

            tly = jnp.maximum(CB[0], rows[0])
            tlx = jnp.maximum(CB[1], rows[1])
            bry = jnp.minimum(CB[2], rows[2])
            brx = jnp.minimum(CB[3], rows[3])
            cap = jnp.where((tly < bry) & (tlx < brx),
                            (bry - tly) * (brx - tlx), 0.0)
            iou = cap / ((CB[4] + rows[4]) - cap)
            s_scr[...] = (iou >= TH).astype(F32)

            valid_row = ((k * B + iotaL) < N_PRE).astype(F32)

            def ibody(i, st):
                keep_row, kf_row, c0 = st
                cur = (iotaL == i).astype(F32)
                curS = (iotaS == i).astype(F32)
                srow = s_scr[pl.ds(i, 1), :]
                within = jnp.max(srow * keep_row)
                supi = jnp.max(sup_col * curS)
                vali = jnp.max(valid_row * cur)
                keep_i = jnp.where(within + supi > 0.0, 0.0, 1.0) * vali
                c1 = c0 + keep_i.astype(I32)
                kf_i = keep_i * (c1 <= N_POST).astype(F32)
                return (keep_row + cur * keep_i, kf_row + cur * kf_i, c1)

            keep_row, kf_row, c_new = lax.fori_loop(
                0, B, ibody,
                (jnp.zeros((1, B), F32), jnp.zeros((1, B), F32), cnt))

            for c in range(5):
                dA[c, pl.ds(k, 1), :] = rows[c] * keep_row
            for c in range(4):
                out[c, pl.ds(k, 1), :] = rows[c] * kf_row
            return c_new

        return lax.cond(count >= N_POST, fast, slow, count)

    lax.fori_loop(0, NB, block_step, jnp.int32(0))


def kernel(anchor, offset, score):
    spad = jnp.zeros((NS,), F32).at[:N].set(score[:, 1]).reshape(SR, SL)
    order_grid = pl.pallas_call(
        _sort_body,
        out_shape=jax.ShapeDtypeStruct((SR, SL), I32),
    )(spad)
    order = order_grid.reshape(NS)[:PAD]

    data = jnp.zeros((N, 128), F32).at[:, :4].set(anchor).at[:, 4:8].set(offset)
    gathered = _sc_gather(data, order)

    ancA = gathered[:, :4].T.reshape(4, NB, B)
    offA = gathered[:, 4:8].T.reshape(4, NB, B)
    out_t = pl.pallas_call(
        _nms_body,
        out_shape=jax.ShapeDtypeStruct((4, NB, B), F32),
        scratch_shapes=[
            pltpu.VMEM((5, NB, B), F32),
            pltpu.VMEM((5, NB, B), F32),
            pltpu.VMEM((B, B), F32),
        ],
    )(ancA, offA)
    return out_t.reshape(4, PAD).T[:N_PRE]


# speedup vs baseline: 2.4289x; 2.4289x over previous
import functools

import jax
import jax.numpy as jnp
from jax import lax
from jax.experimental import pallas as pl
from jax.experimental.pallas import tpu as pltpu
from jax.experimental.pallas import tpu_sc as plsc

N = 20000
N_PRE = 12000
N_POST = 2000
TH = 0.7
B = 128
NB = 94
PAD = NB * B
NS = 32768
SR, SL = 256, 128
NW = 32
BPW = PAD // NW
F32 = jnp.float32
I32 = jnp.int32
U32 = jnp.uint32


def _sort_body(score_ref, ord_ref):
    s = score_ref[...]
    bu = lax.bitcast_convert_type(s, U32)
    li = (lax.broadcasted_iota(I32, (SR, SL), 0) * SL
          + lax.broadcasted_iota(I32, (SR, SL), 1))
    neg = lax.bitcast_convert_type(s, I32) < 0
    ik = jnp.where(neg, bu, ~(bu ^ jnp.uint32(0x80000000)))
    ik = jnp.where(li < N, ik, jnp.uint32(0xFFFFFFFF))
    v = li

    iotaR = lax.broadcasted_iota(I32, (SR, SL), 0)
    iotaL = lax.broadcasted_iota(I32, (SR, SL), 1)

    size = 2
    while size <= NS:
        j = size // 2
        while j >= 1:
            if j >= SL:
                m = j // SL
                kd = jnp.roll(ik, -m, axis=0)
                ku = jnp.roll(ik, m, axis=0)
                vd = jnp.roll(v, -m, axis=0)
                vu = jnp.roll(v, m, axis=0)
                low = (iotaR & m) == 0
            else:
                kd = jnp.roll(ik, -j, axis=1)
                ku = jnp.roll(ik, j, axis=1)
                vd = jnp.roll(v, -j, axis=1)
                vu = jnp.roll(v, j, axis=1)
                low = (iotaL & j) == 0
            kp = jnp.where(low, kd, ku)
            vp = jnp.where(low, vd, vu)
            asc = (li & size) == 0
            take_min = low == asc
            i_is_min = (ik < kp) | ((ik == kp) & (v < vp))
            keep = i_is_min == take_min
            ik = jnp.where(keep, ik, kp)
            v = jnp.where(keep, v, vp)
            j //= 2
        size *= 2
    ord_ref[...] = v


def _gather_body(data_hbm, idx_hbm, out_hbm, idx_v, rows_v, sem):
    wid = lax.axis_index("s") * 2 + lax.axis_index("c")
    base = wid * BPW
    pltpu.sync_copy(idx_hbm.at[pl.ds(base, BPW)], idx_v)
    pltpu.async_copy(data_hbm.at[idx_v], rows_v, sem).wait()
    pltpu.sync_copy(rows_v, out_hbm.at[pl.ds(base, BPW)])


def _sc_gather(data, idx):
    k = functools.partial(
        pl.kernel,
        mesh=plsc.VectorSubcoreMesh(core_axis_name="c", subcore_axis_name="s"),
        out_type=jax.ShapeDtypeStruct((PAD, 128), F32),
        scratch_types=[
            pltpu.VMEM((BPW,), I32),
            pltpu.VMEM((BPW, 128), F32),
            pltpu.SemaphoreType.DMA,
        ],
    )(_gather_body)
    return k(data, idx)


def _decode(aref, oref, bref):
    a0, a1, a2, a3 = aref[0], aref[1], aref[2], aref[3]
    cy = (a0 + a2) / 2
    cx = (a1 + a3) / 2
    h = a2 - a0
    w = a3 - a1
    dy, dx = oref[0], oref[1]
    dh = 0.5 * jnp.exp(oref[2])
    dw = 0.5 * jnp.exp(oref[3])
    y1 = jnp.clip(cy + h * (dy - dh), 0.0, 256.0)
    x1 = jnp.clip(cx + w * (dx - dw), 0.0, 256.0)
    y2 = jnp.clip(cy + h * (dy + dh), 0.0, 256.0)
    x2 = jnp.clip(cx + w * (dx + dw), 0.0, 256.0)
    bref[0] = y1
    bref[1] = x1
    bref[2] = y2
    bref[3] = x2
    bref[4] = (y2 - y1) * (x2 - x1)


def _nms_body(ancA, offA, out, bA, dA):
    _decode(ancA, offA, bA)

    iotaL = lax.broadcasted_iota(I32, (1, B), 1)
    iota2R = lax.broadcasted_iota(I32, (B, B), 0)
    iota2L = lax.broadcasted_iota(I32, (B, B), 1)
    eye = (iota2R == iota2L).astype(F32)
    strict_upper = (iota2R < iota2L).astype(F32)
    cum_lt = (iota2R <= iota2L).astype(F32)

    def block_step(k, count):
        def fast(cnt):
            for c in range(4):
                out[c, pl.ds(k, 1), :] = jnp.zeros((1, B), F32)
            return cnt

        def slow(cnt):
            rows = [bA[c, pl.ds(k, 1), :] for c in range(5)]
            stacked = jnp.concatenate(
                [jnp.broadcast_to(r, (B, B)) for r in rows], axis=1)
            cbs = lax.dot_general(stacked, eye, (((0,), (0,)), ((), ())),
                                  precision=lax.Precision.HIGHEST,
                                  preferred_element_type=F32)
            CB = [cbs[c * B:(c + 1) * B, :] for c in range(5)]

            def pbody(j, sup):
                cj = [dA[c, pl.ds(j, 1), :] for c in range(5)]
                tly = jnp.maximum(CB[0], cj[0])
                tlx = jnp.maximum(CB[1], cj[1])
                bry = jnp.minimum(CB[2], cj[2])
                brx = jnp.minimum(CB[3], cj[3])
                cap = jnp.where((tly < bry) & (tlx < brx),
                                (bry - tly) * (brx - tlx), 0.0)
                iou = cap / ((CB[4] + cj[4]) - cap)
                hit = jnp.max((iou >= TH).astype(F32), axis=1, keepdims=True)
                return jnp.maximum(sup, hit)

            sup_col = lax.fori_loop(0, k, pbody, jnp.zeros((B, 1), F32))

            tly = jnp.maximum(CB[0], rows[0])
            tlx = jnp.maximum(CB[1], rows[1])
            bry = jnp.minimum(CB[2], rows[2])
            brx = jnp.minimum(CB[3], rows[3])
            cap = jnp.where((tly < bry) & (tlx < brx),
                            (bry - tly) * (brx - tlx), 0.0)
            iou = cap / ((CB[4] + rows[4]) - cap)
            conf = (iou >= TH).astype(F32) * strict_upper

            supT = lax.dot_general(jnp.broadcast_to(sup_col, (B, B)), eye,
                                   (((0,), (0,)), ((), ())),
                                   preferred_element_type=F32)
            inv_row = ((k * B + iotaL) >= N_PRE).astype(F32)
            sup0 = jnp.maximum(supT[0:1, :], inv_row)

            def wcond(st):
                keep, prev, it = st
                return jnp.logical_and(jnp.any(keep != prev), it < B + 4)

            def wbody(st):
                keep, prev, it = st
                m = lax.dot_general(keep, conf, (((1,), (0,)), ((), ())),
                                    preferred_element_type=F32)
                knew = jnp.where(sup0 + m > 0.0, 0.0, 1.0)
                return (knew, keep, it + 1)

            keep_row, _, _ = lax.while_loop(
                wcond, wbody,
                (jnp.where(sup0 > 0.0, 0.0, 1.0),
                 jnp.full((1, B), -1.0, F32), jnp.int32(0)))

            ranks = lax.dot_general(keep_row, cum_lt, (((1,), (0,)), ((), ())),
                                    preferred_element_type=F32)
            kf_row = keep_row * ((cnt.astype(F32) + ranks)
                                 <= float(N_POST)).astype(F32)
            c_new = cnt + jnp.max(ranks).astype(I32)

            for c in range(5):
                dA[c, pl.ds(k, 1), :] = rows[c] * keep_row
            for c in range(4):
                out[c, pl.ds(k, 1), :] = rows[c] * kf_row
            return c_new

        return lax.cond(count >= N_POST, fast, slow, count)

    lax.fori_loop(0, NB, block_step, jnp.int32(0))


def kernel(anchor, offset, score):
    spad = jnp.zeros((NS,), F32).at[:N].set(score[:, 1]).reshape(SR, SL)
    order_grid = pl.pallas_call(
        _sort_body,
        out_shape=jax.ShapeDtypeStruct((SR, SL), I32),
    )(spad)
    order = order_grid.reshape(NS)[:PAD]

    data = jnp.zeros((N, 128), F32).at[:, :4].set(anchor).at[:, 4:8].set(offset)
    gathered = _sc_gather(data, order)

    ancA = gathered[:, :4].T.reshape(4, NB, B)
    offA = gathered[:, 4:8].T.reshape(4, NB, B)
    out_t = pl.pallas_call(
        _nms_body,
        out_shape=jax.ShapeDtypeStruct((4, NB, B), F32),
        scratch_shapes=[
            pltpu.VMEM((5, NB, B), F32),
            pltpu.VMEM((5, NB, B), F32),
        ],
    )(ancA, offA)
    return out_t.reshape(4, PAD).T[:N_PRE]


# speedup vs baseline: 3.7880x; 1.5596x over previous
import functools

import jax
import jax.numpy as jnp
from jax import lax
from jax.experimental import pallas as pl
from jax.experimental.pallas import tpu as pltpu
from jax.experimental.pallas import tpu_sc as plsc

N = 20000
N_PRE = 12000
N_POST = 2000
TH = 0.7
B = 128
NB = 94
PAD = NB * B
NS = 32768
SR, SL = 256, 128
NW = 32
BPW = PAD // NW
F32 = jnp.float32
I32 = jnp.int32
U32 = jnp.uint32


def _sort_body(score_ref, ord_ref):
    s = score_ref[...]
    bu = lax.bitcast_convert_type(s, U32)
    li = (lax.broadcasted_iota(I32, (SR, SL), 0) * SL
          + lax.broadcasted_iota(I32, (SR, SL), 1))
    neg = lax.bitcast_convert_type(s, I32) < 0
    ik = jnp.where(neg, bu, ~(bu ^ jnp.uint32(0x80000000)))
    ik = jnp.where(li < N, ik, jnp.uint32(0xFFFFFFFF))
    v = li

    iotaR = lax.broadcasted_iota(I32, (SR, SL), 0)
    iotaL = lax.broadcasted_iota(I32, (SR, SL), 1)

    size = 2
    while size <= NS:
        j = size // 2
        while j >= 1:
            if j >= SL:
                m = j // SL
                kd = jnp.roll(ik, -m, axis=0)
                ku = jnp.roll(ik, m, axis=0)
                vd = jnp.roll(v, -m, axis=0)
                vu = jnp.roll(v, m, axis=0)
                low = (iotaR & m) == 0
            else:
                kd = jnp.roll(ik, -j, axis=1)
                ku = jnp.roll(ik, j, axis=1)
                vd = jnp.roll(v, -j, axis=1)
                vu = jnp.roll(v, j, axis=1)
                low = (iotaL & j) == 0
            kp = jnp.where(low, kd, ku)
            vp = jnp.where(low, vd, vu)
            asc = (li & size) == 0
            take_min = low == asc
            i_is_min = (ik < kp) | ((ik == kp) & (v < vp))
            keep = i_is_min == take_min
            ik = jnp.where(keep, ik, kp)
            v = jnp.where(keep, v, vp)
            j //= 2
        size *= 2
    ord_ref[...] = v


def _gather_body(data_hbm, idx_hbm, out_hbm, idx_v, rows_v, sem):
    wid = lax.axis_index("s") * 2 + lax.axis_index("c")
    base = wid * BPW
    pltpu.sync_copy(idx_hbm.at[pl.ds(base, BPW)], idx_v)
    pltpu.async_copy(data_hbm.at[idx_v], rows_v, sem).wait()
    pltpu.sync_copy(rows_v, out_hbm.at[pl.ds(base, BPW)])


def _sc_gather(data, idx):
    k = functools.partial(
        pl.kernel,
        mesh=plsc.VectorSubcoreMesh(core_axis_name="c", subcore_axis_name="s"),
        out_type=jax.ShapeDtypeStruct((PAD, 16), F32),
        compiler_params=pltpu.CompilerParams(use_tc_tiling_on_sc=False),
        scratch_types=[
            pltpu.VMEM((BPW,), I32),
            pltpu.VMEM((BPW, 16), F32),
            pltpu.SemaphoreType.DMA,
        ],
    )(_gather_body)
    return k(data, idx)


def _decode(aref, oref, bref):
    a0, a1, a2, a3 = aref[0], aref[1], aref[2], aref[3]
    cy = (a0 + a2) / 2
    cx = (a1 + a3) / 2
    h = a2 - a0
    w = a3 - a1
    dy, dx = oref[0], oref[1]
    dh = 0.5 * jnp.exp(oref[2])
    dw = 0.5 * jnp.exp(oref[3])
    y1 = jnp.clip(cy + h * (dy - dh), 0.0, 256.0)
    x1 = jnp.clip(cx + w * (dx - dw), 0.0, 256.0)
    y2 = jnp.clip(cy + h * (dy + dh), 0.0, 256.0)
    x2 = jnp.clip(cx + w * (dx + dw), 0.0, 256.0)
    bref[0] = y1
    bref[1] = x1
    bref[2] = y2
    bref[3] = x2
    bref[4] = (y2 - y1) * (x2 - x1)


def _nms_body(ancA, offA, out, bA, dA):
    _decode(ancA, offA, bA)

    iotaL = lax.broadcasted_iota(I32, (1, B), 1)
    iota2R = lax.broadcasted_iota(I32, (B, B), 0)
    iota2L = lax.broadcasted_iota(I32, (B, B), 1)
    eye = (iota2R == iota2L).astype(F32)
    strict_upper = (iota2R < iota2L).astype(F32)
    cum_lt = (iota2R <= iota2L).astype(F32)

    def block_step(k, count):
        def fast(cnt):
            for c in range(4):
                out[c, pl.ds(k, 1), :] = jnp.zeros((1, B), F32)
            return cnt

        def slow(cnt):
            rows = [bA[c, pl.ds(k, 1), :] for c in range(5)]
            stacked = jnp.concatenate(
                [jnp.broadcast_to(r, (B, B)) for r in rows], axis=1)
            cbs = lax.dot_general(stacked, eye, (((0,), (0,)), ((), ())),
                                  precision=lax.Precision.HIGHEST,
                                  preferred_element_type=F32)
            CB = [cbs[c * B:(c + 1) * B, :] for c in range(5)]

            def pbody(j, sup):
                cj = [dA[c, pl.ds(j, 1), :] for c in range(5)]
                tly = jnp.maximum(CB[0], cj[0])
                tlx = jnp.maximum(CB[1], cj[1])
                bry = jnp.minimum(CB[2], cj[2])
                brx = jnp.minimum(CB[3], cj[3])
                cap = jnp.where((tly < bry) & (tlx < brx),
                                (bry - tly) * (brx - tlx), 0.0)
                iou = cap / ((CB[4] + cj[4]) - cap)
                hit = jnp.max((iou >= TH).astype(F32), axis=1, keepdims=True)
                return jnp.maximum(sup, hit)

            sup_col = lax.fori_loop(0, k, pbody, jnp.zeros((B, 1), F32))

            tly = jnp.maximum(CB[0], rows[0])
            tlx = jnp.maximum(CB[1], rows[1])
            bry = jnp.minimum(CB[2], rows[2])
            brx = jnp.minimum(CB[3], rows[3])
            cap = jnp.where((tly < bry) & (tlx < brx),
                            (bry - tly) * (brx - tlx), 0.0)
            iou = cap / ((CB[4] + rows[4]) - cap)
            conf = (iou >= TH).astype(F32) * strict_upper

            supT = lax.dot_general(jnp.broadcast_to(sup_col, (B, B)), eye,
                                   (((0,), (0,)), ((), ())),
                                   preferred_element_type=F32)
            inv_row = ((k * B + iotaL) >= N_PRE).astype(F32)
            sup0 = jnp.maximum(supT[0:1, :], inv_row)

            def wcond(st):
                keep, prev, it = st
                return jnp.logical_and(jnp.any(keep != prev), it < B + 4)

            def wbody(st):
                keep, prev, it = st
                m = lax.dot_general(keep, conf, (((1,), (0,)), ((), ())),
                                    preferred_element_type=F32)
                knew = jnp.where(sup0 + m > 0.0, 0.0, 1.0)
                return (knew, keep, it + 1)

            keep_row, _, _ = lax.while_loop(
                wcond, wbody,
                (jnp.where(sup0 > 0.0, 0.0, 1.0),
                 jnp.full((1, B), -1.0, F32), jnp.int32(0)))

            ranks = lax.dot_general(keep_row, cum_lt, (((1,), (0,)), ((), ())),
                                    preferred_element_type=F32)
            kf_row = keep_row * ((cnt.astype(F32) + ranks)
                                 <= float(N_POST)).astype(F32)
            c_new = cnt + jnp.max(ranks).astype(I32)

            for c in range(5):
                dA[c, pl.ds(k, 1), :] = rows[c] * keep_row
            for c in range(4):
                out[c, pl.ds(k, 1), :] = rows[c] * kf_row
            return c_new

        return lax.cond(count >= N_POST, fast, slow, count)

    lax.fori_loop(0, NB, block_step, jnp.int32(0))


def kernel(anchor, offset, score):
    spad = jnp.zeros((NS,), F32).at[:N].set(score[:, 1]).reshape(SR, SL)
    order_grid = pl.pallas_call(
        _sort_body,
        out_shape=jax.ShapeDtypeStruct((SR, SL), I32),
    )(spad)
    order = order_grid.reshape(NS)[:PAD]

    data = jnp.concatenate([anchor, offset, jnp.zeros((N, 8), F32)], axis=1)
    gathered = _sc_gather(data, order)

    ancA = gathered[:, :4].T.reshape(4, NB, B)
    offA = gathered[:, 4:8].T.reshape(4, NB, B)
    out_t = pl.pallas_call(
        _nms_body,
        out_shape=jax.ShapeDtypeStruct((4, NB, B), F32),
        scratch_shapes=[
            pltpu.VMEM((5, NB, B), F32),
            pltpu.VMEM((5, NB, B), F32),
        ],
    )(ancA, offA)
    return out_t.reshape(4, PAD).T[:N_PRE]
